# Initial kernel scaffold; baseline (speedup 1.0000x reference)
import jax
import jax.numpy as jnp
from jax.experimental import pallas as pl


def kernel(x, edge_index, W, bparam):
    raise NotImplementedError("write your pallas kernel here")


# speedup vs baseline: 586.4768x; 586.4768x over previous
import functools

import jax
import jax.numpy as jnp
from jax import lax
from jax.experimental import pallas as pl
from jax.experimental.pallas import tpu as pltpu
from jax.experimental.pallas import tpu_sc as plsc

_N, _C, _K, _OUT = 10000, 128, 32, 128
_NW = 32
_NP = 10240
_NODES_PER_W = _NP // _NW
_NB = 4
_CHUNKS = _NODES_PER_W // _NB
_LANES = 16
_CI = _C // _LANES


def _maxdiff_body(x_hbm, idxj_hbm, idxi_hbm, out_hbm,
                  idx_j_v, idx_i_v, rows_j, rows_i, out_v, sem_j, sem_i):
    wid = lax.axis_index("s") * 2 + lax.axis_index("c")
    base_node = wid * _NODES_PER_W

    def chunk_body(g, carry):
        nb = base_node + g * _NB
        pltpu.sync_copy(idxj_hbm.at[pl.ds(nb * _K, _NB * _K)], idx_j_v)
        pltpu.sync_copy(idxi_hbm.at[pl.ds(nb * _K, _NB * _K)], idx_i_v)
        cpj = pltpu.async_copy(x_hbm.at[idx_j_v], rows_j, sem_j)
        cpi = pltpu.async_copy(x_hbm.at[idx_i_v], rows_i, sem_i)
        cpj.wait()
        cpi.wait()
        for n in range(_NB):
            for c in range(_CI):
                sl = pl.ds(c * _LANES, _LANES)

                def kbody(k, acc, n=n, sl=sl):
                    r = n * _K + k
                    return jnp.maximum(acc, rows_j[r, sl] - rows_i[r, sl])

                acc0 = jnp.full((_LANES,), -jnp.inf, dtype=jnp.float32)
                out_v[n, sl] = lax.fori_loop(0, _K, kbody, acc0)
        pltpu.sync_copy(out_v, out_hbm.at[pl.ds(nb, _NB)])
        return carry

    lax.fori_loop(0, _CHUNKS, chunk_body, 0)


_maxdiff_kernel = functools.partial(
    pl.kernel,
    mesh=plsc.VectorSubcoreMesh(core_axis_name="c", subcore_axis_name="s"),
    out_type=jax.ShapeDtypeStruct((_NP, _C), jnp.float32),
    scratch_types=[
        pltpu.VMEM((_NB * _K,), jnp.int32),
        pltpu.VMEM((_NB * _K,), jnp.int32),
        pltpu.VMEM((_NB * _K, _C), jnp.float32),
        pltpu.VMEM((_NB * _K, _C), jnp.float32),
        pltpu.VMEM((_NB, _C), jnp.float32),
        pltpu.SemaphoreType.DMA,
        pltpu.SemaphoreType.DMA,
    ],
)(_maxdiff_body)


_TN = 1024


def _mlp_body(x_ref, md_ref, we_ref, wo_ref, b_ref, o_ref):
    acc = jnp.dot(x_ref[...], we_ref[...], preferred_element_type=jnp.float32)
    acc = acc + jnp.dot(md_ref[...], wo_ref[...],
                        preferred_element_type=jnp.float32)
    o_ref[...] = jnp.maximum(acc + b_ref[...], 0.0)


def kernel(x, edge_index, W, bparam):
    x2 = x[0]
    idx = edge_index[:, 0].astype(jnp.int32)
    idx = jnp.pad(idx, ((0, 0), (0, _NP - _N), (0, 0)))
    idx_j = idx[0].reshape(_NP * _K)
    idx_i = idx[1].reshape(_NP * _K)

    maxdiff = _maxdiff_kernel(x2, idx_j, idx_i)

    xp = jnp.pad(x2, ((0, _NP - _N), (0, 0)))
    we_t = W[:, 0::2].T
    wo_t = W[:, 1::2].T
    b2 = bparam.reshape(1, _OUT)

    out = pl.pallas_call(
        _mlp_body,
        grid=(_NP // _TN,),
        in_specs=[
            pl.BlockSpec((_TN, _C), lambda i: (i, 0)),
            pl.BlockSpec((_TN, _C), lambda i: (i, 0)),
            pl.BlockSpec((_C, _OUT), lambda i: (0, 0)),
            pl.BlockSpec((_C, _OUT), lambda i: (0, 0)),
            pl.BlockSpec((1, _OUT), lambda i: (0, 0)),
        ],
        out_specs=pl.BlockSpec((_TN, _OUT), lambda i: (i, 0)),
        out_shape=jax.ShapeDtypeStruct((_NP, _OUT), jnp.float32),
    )(xp, maxdiff, we_t, wo_t, b2)

    return out[:_N][None]


# speedup vs baseline: 790.2934x; 1.3475x over previous
import functools

import jax
import jax.numpy as jnp
from jax import lax
from jax.experimental import pallas as pl
from jax.experimental.pallas import tpu as pltpu
from jax.experimental.pallas import tpu_sc as plsc

_N, _C, _K, _OUT = 10000, 128, 32, 128
_NW = 32
_NP = 10240
_NODES_PER_W = _NP // _NW
_NB = 4
_CIDX = _NB * _K
_CHUNKS = _NODES_PER_W // _NB
_NBUF = 2
_SEGS = _CHUNKS // _NBUF
_LANES = 16
_CI = _C // _LANES
_NEG_INF = float("-inf")


def _maxdiff_body(x_hbm, idxj_hbm, idxi_hbm, out_hbm,
                  idxj_v, idxi_v, rows_j, rows_i, out_v,
                  sem_g, sem_o):
    wid = lax.axis_index("s") * 2 + lax.axis_index("c")
    chunk0 = wid * _CHUNKS

    pltpu.sync_copy(idxj_hbm.at[pl.ds(chunk0, _CHUNKS)], idxj_v)
    pltpu.sync_copy(idxi_hbm.at[pl.ds(chunk0, _CHUNKS)], idxi_v)

    def gather(buf, g):
        pltpu.async_copy(x_hbm.at[idxj_v.at[g]], rows_j.at[buf], sem_g.at[buf])
        pltpu.async_copy(x_hbm.at[idxi_v.at[g]], rows_i.at[buf], sem_g.at[buf])

    def drain(buf):
        pltpu.make_async_copy(x_hbm.at[idxj_v.at[0]], rows_j.at[buf],
                              sem_g.at[buf]).wait()
        pltpu.make_async_copy(x_hbm.at[idxi_v.at[0]], rows_i.at[buf],
                              sem_g.at[buf]).wait()

    for b in range(_NBUF):
        gather(b, b)

    def seg_body(s, carry):
        for b in range(_NBUF):
            g = s * _NBUF + b
            drain(b)
            @pl.when(s > 0)
            def _():
                pltpu.make_async_copy(out_v.at[b],
                                      out_hbm.at[pl.ds(0, _NB)],
                                      sem_o.at[b]).wait()
            for n in range(_NB):
                accs = tuple(jnp.full((_LANES,), _NEG_INF, dtype=jnp.float32)
                             for _ in range(_CI))

                def kbody(k, accs, n=n, b=b):
                    r = n * _K + k
                    return tuple(
                        jnp.maximum(accs[c],
                                    rows_j[b, r, pl.ds(c * _LANES, _LANES)]
                                    - rows_i[b, r, pl.ds(c * _LANES, _LANES)])
                        for c in range(_CI))

                accs = lax.fori_loop(0, _K, kbody, accs)
                for c in range(_CI):
                    out_v[b, n, pl.ds(c * _LANES, _LANES)] = accs[c]
            nb0 = (chunk0 + g) * _NB
            pltpu.async_copy(out_v.at[b], out_hbm.at[pl.ds(nb0, _NB)],
                             sem_o.at[b])
            @pl.when(g + _NBUF < _CHUNKS)
            def _(g=g, b=b):
                gather(b, g + _NBUF)
        return carry

    lax.fori_loop(0, _SEGS, seg_body, 0)
    for b in range(_NBUF):
        pltpu.make_async_copy(out_v.at[b], out_hbm.at[pl.ds(0, _NB)],
                              sem_o.at[b]).wait()


_maxdiff_kernel = functools.partial(
    pl.kernel,
    mesh=plsc.VectorSubcoreMesh(core_axis_name="c", subcore_axis_name="s"),
    out_type=jax.ShapeDtypeStruct((_NP, _C), jnp.float32),
    scratch_types=[
        pltpu.VMEM((_CHUNKS, _CIDX), jnp.int32),
        pltpu.VMEM((_CHUNKS, _CIDX), jnp.int32),
        pltpu.VMEM((_NBUF, _CIDX, _C), jnp.float32),
        pltpu.VMEM((_NBUF, _CIDX, _C), jnp.float32),
        pltpu.VMEM((_NBUF, _NB, _C), jnp.float32),
        pltpu.SemaphoreType.DMA((_NBUF,)),
        pltpu.SemaphoreType.DMA((_NBUF,)),
    ],
)(_maxdiff_body)


_TN = 1024


def _mlp_body(x_ref, md_ref, we_ref, wo_ref, b_ref, o_ref):
    acc = jnp.dot(x_ref[...], we_ref[...], preferred_element_type=jnp.float32)
    acc = acc + jnp.dot(md_ref[...], wo_ref[...],
                        preferred_element_type=jnp.float32)
    o_ref[...] = jnp.maximum(acc + b_ref[...], 0.0)


def kernel(x, edge_index, W, bparam):
    x2 = x[0]
    idx = edge_index[:, 0].astype(jnp.int32)
    idx = jnp.pad(idx, ((0, 0), (0, _NP - _N), (0, 0)))
    idx_j = idx[0].reshape(_NP * _K // _CIDX, _CIDX)
    idx_i = idx[1].reshape(_NP * _K // _CIDX, _CIDX)

    maxdiff = _maxdiff_kernel(x2, idx_j, idx_i)

    xp = jnp.pad(x2, ((0, _NP - _N), (0, 0)))
    we_t = W[:, 0::2].T
    wo_t = W[:, 1::2].T
    b2 = bparam.reshape(1, _OUT)

    out = pl.pallas_call(
        _mlp_body,
        grid=(_NP // _TN,),
        in_specs=[
            pl.BlockSpec((_TN, _C), lambda i: (i, 0)),
            pl.BlockSpec((_TN, _C), lambda i: (i, 0)),
            pl.BlockSpec((_C, _OUT), lambda i: (0, 0)),
            pl.BlockSpec((_C, _OUT), lambda i: (0, 0)),
            pl.BlockSpec((1, _OUT), lambda i: (0, 0)),
        ],
        out_specs=pl.BlockSpec((_TN, _OUT), lambda i: (i, 0)),
        out_shape=jax.ShapeDtypeStruct((_NP, _OUT), jnp.float32),
    )(xp, maxdiff, we_t, wo_t, b2)

    return out[:_N][None]
